# baseline (speedup 1.0000x reference)
import functools

import jax
import jax.numpy as jnp
from jax.experimental import pallas as pl
from jax.experimental.pallas import tpu as pltpu


def _ffn_body(xe_ref, w1_ref, b1_ref, w2_ref, b2_ref, out_ref):
    j = pl.program_id(1)
    nj = pl.num_programs(1)
    xb = xe_ref[0].astype(jnp.bfloat16)
    w1b = w1_ref[0].astype(jnp.bfloat16)
    h = jax.lax.dot_general(
        xb, w1b, (((1,), (1,)), ((), ())),
        preferred_element_type=jnp.float32)
    h = h + b1_ref[0, 0, 0][None, :]
    h = 0.5 * h * (1.0 + jax.lax.erf(h * 0.7071067811865476))
    w2b = w2_ref[0].astype(jnp.bfloat16)
    part = jax.lax.dot_general(
        h.astype(jnp.bfloat16), w2b, (((1,), (1,)), ((), ())),
        preferred_element_type=jnp.float32)

    @pl.when(j == 0)
    def _():
        out_ref[0] = part

    @pl.when(j != 0)
    def _():
        out_ref[0] += part

    @pl.when(j == nj - 1)
    def _():
        out_ref[0] += b2_ref[0, 0][None, :]


def _ffn(xe, w1, b1r, w2, b2r, *, dt):
    e, c, d = xe.shape
    dff = w1.shape[1]
    nj = dff // dt
    return pl.pallas_call(
        _ffn_body,
        grid=(e, nj),
        in_specs=[
            pl.BlockSpec((1, c, d), lambda i, j: (i, 0, 0)),
            pl.BlockSpec((1, dt, d), lambda i, j: (i, j, 0)),
            pl.BlockSpec((1, 1, 1, dt), lambda i, j: (i, j, 0, 0)),
            pl.BlockSpec((1, d, dt), lambda i, j: (i, 0, j)),
            pl.BlockSpec((1, 1, d), lambda i, j: (i, 0, 0)),
        ],
        out_specs=pl.BlockSpec((1, c, d), lambda i, j: (i, 0, 0)),
        out_shape=jax.ShapeDtypeStruct((e, c, d), jnp.float32),
        compiler_params=pltpu.CompilerParams(
            dimension_semantics=("arbitrary", "arbitrary"),
        ),
    )(xe, w1, b1r, w2, b2r)


def kernel(x, Wr, w1, b1, w2, b2):
    b, s, d = x.shape
    e = Wr.shape[0]
    dff = w1.shape[1]
    x_flat = x.reshape(-1, d)
    n_tok = x_flat.shape[0]
    cap = int(1.25 * n_tok / e)

    router_logits = x_flat @ Wr.T
    router_probs = jax.nn.softmax(router_logits, axis=-1)
    gates = jnp.max(router_probs, axis=-1)
    indices = jnp.argmax(router_probs, axis=-1)

    expert_mask = jax.nn.one_hot(indices, e, dtype=jnp.float32)
    density = expert_mask.mean(axis=0)
    prob_mean = router_probs.mean(axis=0)
    load_balance_loss = e * jnp.sum(density * prob_mean) * 0.01
    router_z_loss = jnp.mean(
        jax.scipy.special.logsumexp(router_probs, axis=-1)) * 0.001
    aux_loss = load_balance_loss + router_z_loss

    tok = jnp.arange(n_tok, dtype=jnp.int32)
    idx32 = indices.astype(jnp.int32)
    sorted_e, _, sorted_tok = jax.lax.sort(
        (idx32, -gates, tok), num_keys=2, is_stable=True)
    counts = jnp.sum(expert_mask, axis=0).astype(jnp.int32)
    seg_start = jnp.concatenate(
        [jnp.zeros((1,), jnp.int32), jnp.cumsum(counts)[:-1].astype(jnp.int32)])
    pos = tok - seg_start[sorted_e]
    keep = pos < cap
    fslot = sorted_e * cap + pos
    slot_of_tok = jnp.full((n_tok,), -1, jnp.int32).at[sorted_tok].set(
        jnp.where(keep, fslot, -1))
    f_or_dummy = jnp.where(keep, fslot, e * cap)
    sel_idx = jnp.zeros((e * cap + 1,), jnp.int32).at[f_or_dummy].set(
        sorted_tok)[: e * cap]

    xe = x_flat[sel_idx].reshape(e, cap, d)
    eo = _ffn(xe, w1, b1.reshape(e, -1, 1, 1024), w2, b2.reshape(e, 1, d),
              dt=1024)
    eo_flat = eo.reshape(e * cap, d)
    comb_scale = jnp.where(slot_of_tok >= 0, gates, 0.0)
    src = jnp.maximum(slot_of_tok, 0)
    out_flat = eo_flat[src] * comb_scale[:, None]
    return out_flat.reshape(b, s, d), aux_loss


# speedup vs baseline: 1.0673x; 1.0673x over previous
import functools

import jax
import jax.numpy as jnp
from jax.experimental import pallas as pl
from jax.experimental.pallas import tpu as pltpu


def _ffn_body(xe_ref, w1_ref, b1_ref, w2_ref, b2_ref, out_ref, acc_ref):
    j = pl.program_id(1)
    nj = pl.num_programs(1)
    xb = xe_ref[0]
    w1b = w1_ref[0].astype(jnp.bfloat16)
    h = jax.lax.dot_general(
        xb, w1b, (((1,), (1,)), ((), ())),
        preferred_element_type=jnp.float32)
    h = h + b1_ref[0, 0, 0][None, :]
    h = 0.5 * h * (1.0 + jax.lax.erf(h * 0.7071067811865476))
    w2b = w2_ref[0].astype(jnp.bfloat16)
    part = jax.lax.dot_general(
        h.astype(jnp.bfloat16), w2b, (((1,), (1,)), ((), ())),
        preferred_element_type=jnp.float32)

    @pl.when(j == 0)
    def _():
        acc_ref[...] = part

    @pl.when(j != 0)
    def _():
        acc_ref[...] += part

    @pl.when(j == nj - 1)
    def _():
        out_ref[0] = (acc_ref[...] + b2_ref[0, 0][None, :]).astype(
            jnp.bfloat16)


def _ffn(xe, w1, b1r, w2, b2r, *, dt):
    e, c, d = xe.shape
    dff = w1.shape[1]
    nj = dff // dt
    return pl.pallas_call(
        _ffn_body,
        grid=(e, nj),
        in_specs=[
            pl.BlockSpec((1, c, d), lambda i, j: (i, 0, 0)),
            pl.BlockSpec((1, dt, d), lambda i, j: (i, j, 0)),
            pl.BlockSpec((1, 1, 1, dt), lambda i, j: (i, j, 0, 0)),
            pl.BlockSpec((1, d, dt), lambda i, j: (i, 0, j)),
            pl.BlockSpec((1, 1, d), lambda i, j: (i, 0, 0)),
        ],
        out_specs=pl.BlockSpec((1, c, d), lambda i, j: (i, 0, 0)),
        out_shape=jax.ShapeDtypeStruct((e, c, d), jnp.bfloat16),
        scratch_shapes=[pltpu.VMEM((c, d), jnp.float32)],
        compiler_params=pltpu.CompilerParams(
            dimension_semantics=("arbitrary", "arbitrary"),
        ),
    )(xe, w1, b1r, w2, b2r)


def kernel(x, Wr, w1, b1, w2, b2):
    b, s, d = x.shape
    e = Wr.shape[0]
    dff = w1.shape[1]
    x_flat = x.reshape(-1, d)
    n_tok = x_flat.shape[0]
    cap = int(1.25 * n_tok / e)

    router_logits = x_flat @ Wr.T
    router_probs = jax.nn.softmax(router_logits, axis=-1)
    gates = jnp.max(router_probs, axis=-1)
    indices = jnp.argmax(router_probs, axis=-1)

    expert_mask = jax.nn.one_hot(indices, e, dtype=jnp.float32)
    density = expert_mask.mean(axis=0)
    prob_mean = router_probs.mean(axis=0)
    load_balance_loss = e * jnp.sum(density * prob_mean) * 0.01
    router_z_loss = jnp.mean(
        jax.scipy.special.logsumexp(router_probs, axis=-1)) * 0.001
    aux_loss = load_balance_loss + router_z_loss

    tok = jnp.arange(n_tok, dtype=jnp.int32)
    idx32 = indices.astype(jnp.int32)
    sorted_e, _, sorted_tok = jax.lax.sort(
        (idx32, -gates, tok), num_keys=2, is_stable=True)
    counts = jnp.sum(expert_mask, axis=0).astype(jnp.int32)
    seg_start = jnp.concatenate(
        [jnp.zeros((1,), jnp.int32), jnp.cumsum(counts)[:-1].astype(jnp.int32)])
    pos = tok - seg_start[sorted_e]
    keep = pos < cap
    fslot = sorted_e * cap + pos
    slot_of_tok = jnp.full((n_tok,), -1, jnp.int32).at[sorted_tok].set(
        jnp.where(keep, fslot, -1))
    f_or_dummy = jnp.where(keep, fslot, e * cap)
    sel_idx = jnp.zeros((e * cap + 1,), jnp.int32).at[f_or_dummy].set(
        sorted_tok)[: e * cap]

    x_bf = x_flat.astype(jnp.bfloat16)
    xe = x_bf[sel_idx].reshape(e, cap, d)
    eo = _ffn(xe, w1, b1.reshape(e, -1, 1, 1024), w2, b2.reshape(e, 1, d),
              dt=1024)
    eo_flat = eo.reshape(e * cap, d)
    comb_scale = jnp.where(slot_of_tok >= 0, gates, 0.0)
    src = jnp.maximum(slot_of_tok, 0)
    out_flat = eo_flat[src].astype(jnp.float32) * comb_scale[:, None]
    return out_flat.reshape(b, s, d), aux_loss
